# baseline (speedup 1.0000x reference)
import functools

import jax
import jax.numpy as jnp
from jax import lax
from jax.experimental import pallas as pl
from jax.experimental.pallas import tpu as pltpu
from jax.experimental.pallas import tpu_sc as plsc

NC = 2
NS = 16
NW = NC * NS
CHUNK = 128
K = 2
SUB = 4
ZB = 128
NSTAGE = 2


def _make_sc_agg(n, d, cpt, zr):
    aggr = NS * zr
    spc = cpt // NSTAGE
    mesh = plsc.VectorSubcoreMesh(core_axis_name="c", subcore_axis_name="s")

    @functools.partial(
        pl.kernel,
        out_type=jax.ShapeDtypeStruct((NC * aggr, d), jnp.float32),
        mesh=mesh,
        scratch_types=[
            pltpu.VMEM((spc, CHUNK), jnp.int32),
            pltpu.VMEM((spc, CHUNK), jnp.int32),
            pltpu.VMEM((K * CHUNK, d), jnp.float32),
            pltpu.VMEM_SHARED((aggr, d), jnp.float32),
            pltpu.SemaphoreType.DMA,
            pltpu.SemaphoreType.DMA,
        ],
    )
    def sc_agg(x_hbm, srcw_hbm, dstw_hbm, zero_hbm, out_hbm,
               src_v, dst_v, rows_v, agg_sh, gsem, ssem):
        cid = lax.axis_index("c")
        sid = lax.axis_index("s")
        wid = cid * NS + sid

        pltpu.sync_copy(zero_hbm, rows_v.at[pl.ds(0, ZB)])
        zbase = sid * zr
        nfull, rem = zr // ZB, zr % ZB
        for t in range(nfull):
            pltpu.sync_copy(rows_v.at[pl.ds(0, ZB)],
                            agg_sh.at[pl.ds(zbase + t * ZB, ZB)])
        if rem:
            pltpu.sync_copy(rows_v.at[pl.ds(0, rem)],
                            agg_sh.at[pl.ds(zbase + nfull * ZB, rem)])
        plsc.subcore_barrier()

        sr = CHUNK // SUB
        def body(p, carry):
            base = p * K
            gs = [pltpu.async_copy(
                      x_hbm.at[src_v.at[base + k, pl.ds(q * sr, sr)]],
                      rows_v.at[pl.ds((k * SUB + q) * sr, sr)], gsem)
                  for k in range(K) for q in range(SUB)]
            for g in gs:
                g.wait()
            ss = [pltpu.async_copy(rows_v.at[pl.ds(k * CHUNK, CHUNK)],
                                   agg_sh.at[dst_v.at[base + k]], ssem,
                                   add=True)
                  for k in range(K)]
            for s_ in ss:
                s_.wait()
            return carry

        for s in range(NSTAGE):
            pltpu.sync_copy(srcw_hbm.at[wid, pl.ds(s * spc, spc)], src_v)
            pltpu.sync_copy(dstw_hbm.at[wid, pl.ds(s * spc, spc)], dst_v)
            lax.fori_loop(0, spc // K, body, 0)
        plsc.subcore_barrier()

        obase = cid * aggr + zbase
        for t in range(nfull):
            pltpu.sync_copy(agg_sh.at[pl.ds(zbase + t * ZB, ZB)],
                            rows_v.at[pl.ds(0, ZB)])
            pltpu.sync_copy(rows_v.at[pl.ds(0, ZB)],
                            out_hbm.at[pl.ds(obase + t * ZB, ZB)])
        if rem:
            pltpu.sync_copy(agg_sh.at[pl.ds(zbase + nfull * ZB, rem)],
                            rows_v.at[pl.ds(0, rem)])
            pltpu.sync_copy(rows_v.at[pl.ds(0, rem)],
                            out_hbm.at[pl.ds(obase + nfull * ZB, rem)])

    return sc_agg, aggr


def _mlp_body(n, x_ref, p_ref, w1_ref, w1t_ref, b1_ref, gamma_ref, beta_ref,
              w2_ref, w2t_ref, b2_ref, u1_ref, u2_ref, out_ref):
    f32 = jnp.float32
    hi = lax.Precision.HIGHEST

    h = x_ref[...] + p_ref[0, :n, :] + p_ref[1, :n, :]

    u1 = u1_ref[...]
    v1 = jnp.dot(u1, w1_ref[...], precision=hi, preferred_element_type=f32)
    v1 = v1 / (jnp.sqrt(jnp.sum(v1 * v1)) + 1e-12)
    wv1 = jnp.dot(v1, w1t_ref[...], precision=hi, preferred_element_type=f32)
    sigma1 = jnp.sum(wv1 * wv1) / (jnp.sqrt(jnp.sum(wv1 * wv1)) + 1e-12)

    h1 = jnp.dot(h, w1t_ref[...], precision=hi, preferred_element_type=f32)
    h1 = h1 / sigma1 + b1_ref[...]
    h1 = jnp.maximum(h1, 0.0)

    mean = jnp.mean(h1, axis=0, keepdims=True)
    var = jnp.mean((h1 - mean) * (h1 - mean), axis=0, keepdims=True)
    hn = (h1 - mean) / jnp.sqrt(var + 1e-5) * gamma_ref[...] + beta_ref[...]

    u2 = u2_ref[...]
    v2 = jnp.dot(u2, w2_ref[...], precision=hi, preferred_element_type=f32)
    v2 = v2 / (jnp.sqrt(jnp.sum(v2 * v2)) + 1e-12)
    wv2 = jnp.dot(v2, w2t_ref[...], precision=hi, preferred_element_type=f32)
    sigma2 = jnp.sum(wv2 * wv2) / (jnp.sqrt(jnp.sum(wv2 * wv2)) + 1e-12)

    o = jnp.dot(hn, w2t_ref[...], precision=hi, preferred_element_type=f32)
    out_ref[...] = o / sigma2 + b2_ref[...]


def kernel(x, edge_index, W1, b1, u1, gamma, beta, W2, b2, u2):
    n, d = x.shape
    e = edge_index.shape[1]
    nhid = W1.shape[0]

    cpt = -(-e // (NW * CHUNK))
    cpt = -(-cpt // (NSTAGE * K)) * (NSTAGE * K)
    e_pad = NW * cpt * CHUNK
    zr = -(-(n + 1) // NS)
    zr = -(-zr // 8) * 8

    src = edge_index[0].astype(jnp.int32)
    dst = edge_index[1].astype(jnp.int32)
    pad = e_pad - e
    srcw = jnp.concatenate([src, jnp.zeros((pad,), jnp.int32)]).reshape(
        NW, cpt, CHUNK)
    dst_pad = n + jnp.arange(pad, dtype=jnp.int32) % jnp.int32(NS * zr - n)
    dstw = jnp.concatenate([dst, dst_pad]).reshape(NW, cpt, CHUNK)
    zero = jnp.zeros((ZB, d), jnp.float32)

    sc_agg, aggr = _make_sc_agg(n, d, cpt, zr)
    partials = sc_agg(x, srcw, dstw, zero)
    p = partials.reshape(NC, aggr, d)

    vspec = pl.BlockSpec(memory_space=pltpu.VMEM)
    out = pl.pallas_call(
        functools.partial(_mlp_body, n),
        out_shape=jax.ShapeDtypeStruct((n, nhid), jnp.float32),
        in_specs=[vspec] * 12,
        out_specs=vspec,
    )(x, p, W1, W1.T, b1.reshape(1, nhid), gamma.reshape(1, nhid),
      beta.reshape(1, nhid), W2, W2.T, b2.reshape(1, nhid),
      u1.reshape(1, nhid), u2.reshape(1, nhid))
    return out


# speedup vs baseline: 1.7794x; 1.7794x over previous
import functools

import jax
import jax.numpy as jnp
from jax import lax
from jax.experimental import pallas as pl
from jax.experimental.pallas import tpu as pltpu
from jax.experimental.pallas import tpu_sc as plsc

NC = 2
NS = 16
NW = NC * NS
CHUNK = 128
K = 2
SUB = 4
ZB = 128
NSTAGE = 2


def _row_chunks(total):
    out = [(t * ZB, ZB) for t in range(total // ZB)]
    if total % ZB:
        out.append(((total // ZB) * ZB, total % ZB))
    return out


def _make_sc_gather(d, cpt, zr):
    xrows = NS * zr
    spc = cpt // NSTAGE
    ept = cpt * CHUNK
    mesh = plsc.VectorSubcoreMesh(core_axis_name="c", subcore_axis_name="s")

    @functools.partial(
        pl.kernel,
        out_type=jax.ShapeDtypeStruct((NW * ept, d), jnp.float32),
        mesh=mesh,
        scratch_types=[
            pltpu.VMEM((spc, CHUNK), jnp.int32),
            pltpu.VMEM((K * CHUNK, d), jnp.float32),
            pltpu.VMEM_SHARED((xrows, d), jnp.float32),
            pltpu.SemaphoreType.DMA,
            pltpu.SemaphoreType.DMA,
        ],
    )
    def sc_gather(x_hbm, srcw_hbm, out_hbm, src_v, rows_v, x_sh, gsem, wsem):
        cid = lax.axis_index("c")
        sid = lax.axis_index("s")
        wid = cid * NS + sid

        xbase = sid * zr
        for off, ln in _row_chunks(zr):
            pltpu.sync_copy(x_hbm.at[pl.ds(xbase + off, ln)],
                            rows_v.at[pl.ds(0, ln)])
            pltpu.sync_copy(rows_v.at[pl.ds(0, ln)],
                            x_sh.at[pl.ds(xbase + off, ln)])
        plsc.subcore_barrier()

        ebase = wid * ept
        sr = CHUNK // SUB

        def body(p, carry):
            base = p * K
            gs = [pltpu.async_copy(
                      x_sh.at[src_v.at[base + k, pl.ds(q * sr, sr)]],
                      rows_v.at[pl.ds((k * SUB + q) * sr, sr)], gsem)
                  for k in range(K) for q in range(SUB)]
            for g in gs:
                g.wait()
            ws = [pltpu.async_copy(
                      rows_v.at[pl.ds(k * CHUNK, CHUNK)],
                      out_hbm.at[pl.ds(ebase + (base + k) * CHUNK, CHUNK)],
                      wsem)
                  for k in range(K)]
            for w in ws:
                w.wait()
            return carry

        for s in range(NSTAGE):
            pltpu.sync_copy(srcw_hbm.at[wid, pl.ds(s * spc, spc)], src_v)
            lax.fori_loop(0, spc // K, body, 0)

    return sc_gather


def _make_sc_scatter(d, cpt, zr):
    aggr = NS * zr
    spc = cpt // NSTAGE
    ept = cpt * CHUNK
    mesh = plsc.VectorSubcoreMesh(core_axis_name="c", subcore_axis_name="s")

    @functools.partial(
        pl.kernel,
        out_type=jax.ShapeDtypeStruct((NC * aggr, d), jnp.float32),
        mesh=mesh,
        scratch_types=[
            pltpu.VMEM((spc, CHUNK), jnp.int32),
            pltpu.VMEM((K * CHUNK, d), jnp.float32),
            pltpu.VMEM_SHARED((aggr, d), jnp.float32),
            pltpu.SemaphoreType.DMA,
            pltpu.SemaphoreType.DMA,
        ],
    )
    def sc_scatter(rows_hbm, dstw_hbm, zero_hbm, out_hbm,
                   dst_v, rows_v, agg_sh, gsem, ssem):
        cid = lax.axis_index("c")
        sid = lax.axis_index("s")
        wid = cid * NS + sid

        pltpu.sync_copy(zero_hbm, rows_v.at[pl.ds(0, ZB)])
        zbase = sid * zr
        for off, ln in _row_chunks(zr):
            pltpu.sync_copy(rows_v.at[pl.ds(0, ln)],
                            agg_sh.at[pl.ds(zbase + off, ln)])
        plsc.subcore_barrier()

        ebase = wid * ept

        def body(p, carry):
            base = p * K
            gs = [pltpu.async_copy(
                      rows_hbm.at[pl.ds(ebase + (base + k) * CHUNK, CHUNK)],
                      rows_v.at[pl.ds(k * CHUNK, CHUNK)], gsem)
                  for k in range(K)]
            for g in gs:
                g.wait()
            ss = [pltpu.async_copy(rows_v.at[pl.ds(k * CHUNK, CHUNK)],
                                   agg_sh.at[dst_v.at[base + k]], ssem,
                                   add=True)
                  for k in range(K)]
            for s_ in ss:
                s_.wait()
            return carry

        for s in range(NSTAGE):
            pltpu.sync_copy(dstw_hbm.at[wid, pl.ds(s * spc, spc)], dst_v)
            lax.fori_loop(0, spc // K, body, 0)
        plsc.subcore_barrier()

        obase = cid * aggr + zbase
        for off, ln in _row_chunks(zr):
            pltpu.sync_copy(agg_sh.at[pl.ds(zbase + off, ln)],
                            rows_v.at[pl.ds(0, ln)])
            pltpu.sync_copy(rows_v.at[pl.ds(0, ln)],
                            out_hbm.at[pl.ds(obase + off, ln)])

    return sc_scatter


def _mlp_body(n, x_ref, p_ref, w1_ref, w1t_ref, b1_ref, gamma_ref, beta_ref,
              w2_ref, w2t_ref, b2_ref, u1_ref, u2_ref, out_ref):
    f32 = jnp.float32
    hi = lax.Precision.HIGHEST

    h = x_ref[...] + p_ref[0, :n, :] + p_ref[1, :n, :]

    u1 = u1_ref[...]
    v1 = jnp.dot(u1, w1_ref[...], precision=hi, preferred_element_type=f32)
    v1 = v1 / (jnp.sqrt(jnp.sum(v1 * v1)) + 1e-12)
    wv1 = jnp.dot(v1, w1t_ref[...], precision=hi, preferred_element_type=f32)
    sigma1 = jnp.sum(wv1 * wv1) / (jnp.sqrt(jnp.sum(wv1 * wv1)) + 1e-12)

    h1 = jnp.dot(h, w1t_ref[...], precision=hi, preferred_element_type=f32)
    h1 = h1 / sigma1 + b1_ref[...]
    h1 = jnp.maximum(h1, 0.0)

    mean = jnp.mean(h1, axis=0, keepdims=True)
    var = jnp.mean((h1 - mean) * (h1 - mean), axis=0, keepdims=True)
    hn = (h1 - mean) / jnp.sqrt(var + 1e-5) * gamma_ref[...] + beta_ref[...]

    u2 = u2_ref[...]
    v2 = jnp.dot(u2, w2_ref[...], precision=hi, preferred_element_type=f32)
    v2 = v2 / (jnp.sqrt(jnp.sum(v2 * v2)) + 1e-12)
    wv2 = jnp.dot(v2, w2t_ref[...], precision=hi, preferred_element_type=f32)
    sigma2 = jnp.sum(wv2 * wv2) / (jnp.sqrt(jnp.sum(wv2 * wv2)) + 1e-12)

    o = jnp.dot(hn, w2t_ref[...], precision=hi, preferred_element_type=f32)
    out_ref[...] = o / sigma2 + b2_ref[...]


def kernel(x, edge_index, W1, b1, u1, gamma, beta, W2, b2, u2):
    n, d = x.shape
    e = edge_index.shape[1]
    nhid = W1.shape[0]

    cpt = -(-e // (NW * CHUNK))
    cpt = -(-cpt // (NSTAGE * K)) * (NSTAGE * K)
    e_pad = NW * cpt * CHUNK
    zr = -(-(n + 1) // NS)
    zr = -(-zr // 8) * 8

    src = edge_index[0].astype(jnp.int32)
    dst = edge_index[1].astype(jnp.int32)
    pad = e_pad - e
    srcw = jnp.concatenate([src, jnp.zeros((pad,), jnp.int32)]).reshape(
        NW, cpt, CHUNK)
    dst_pad = n + jnp.arange(pad, dtype=jnp.int32) % jnp.int32(NS * zr - n)
    dstw = jnp.concatenate([dst, dst_pad]).reshape(NW, cpt, CHUNK)
    zero = jnp.zeros((ZB, d), jnp.float32)
    xp = jnp.concatenate([x, jnp.zeros((NS * zr - n, d), jnp.float32)])

    sc_gather = _make_sc_gather(d, cpt, zr)
    rows = sc_gather(xp, srcw)
    sc_scatter = _make_sc_scatter(d, cpt, zr)
    partials = sc_scatter(rows, dstw, zero)
    aggr = NS * zr
    p = partials.reshape(NC, aggr, d)

    vspec = pl.BlockSpec(memory_space=pltpu.VMEM)
    out = pl.pallas_call(
        functools.partial(_mlp_body, n),
        out_shape=jax.ShapeDtypeStruct((n, nhid), jnp.float32),
        in_specs=[vspec] * 12,
        out_specs=vspec,
    )(x, p, W1, W1.T, b1.reshape(1, nhid), gamma.reshape(1, nhid),
      beta.reshape(1, nhid), W2, W2.T, b2.reshape(1, nhid),
      u1.reshape(1, nhid), u2.reshape(1, nhid))
    return out


# speedup vs baseline: 1.7921x; 1.0071x over previous
import functools

import jax
import jax.numpy as jnp
from jax import lax
from jax.experimental import pallas as pl
from jax.experimental.pallas import tpu as pltpu
from jax.experimental.pallas import tpu_sc as plsc

NC = 2
NS = 16
NW = NC * NS
CHUNK = 128
K = 2
SUB = 4
ZB = 128
NSTAGE = 1


def _row_chunks(total):
    out = [(t * ZB, ZB) for t in range(total // ZB)]
    if total % ZB:
        out.append(((total // ZB) * ZB, total % ZB))
    return out


def _make_sc_gather(d, cpt, zr):
    xrows = NS * zr
    spc = cpt // NSTAGE
    ept = cpt * CHUNK
    mesh = plsc.VectorSubcoreMesh(core_axis_name="c", subcore_axis_name="s")

    @functools.partial(
        pl.kernel,
        out_type=jax.ShapeDtypeStruct((NW * ept, d), jnp.float32),
        mesh=mesh,
        scratch_types=[
            pltpu.VMEM((spc, CHUNK), jnp.int32),
            pltpu.VMEM((K * CHUNK, d), jnp.float32),
            pltpu.VMEM_SHARED((xrows, d), jnp.float32),
            pltpu.SemaphoreType.DMA,
            pltpu.SemaphoreType.DMA,
        ],
    )
    def sc_gather(x_hbm, srcw_hbm, out_hbm, src_v, rows_v, x_sh, gsem, wsem):
        cid = lax.axis_index("c")
        sid = lax.axis_index("s")
        wid = cid * NS + sid

        xbase = sid * zr
        for off, ln in _row_chunks(zr):
            pltpu.sync_copy(x_hbm.at[pl.ds(xbase + off, ln)],
                            rows_v.at[pl.ds(0, ln)])
            pltpu.sync_copy(rows_v.at[pl.ds(0, ln)],
                            x_sh.at[pl.ds(xbase + off, ln)])
        plsc.subcore_barrier()

        ebase = wid * ept
        sr = CHUNK // SUB

        def body(p, carry):
            base = p * K
            gs = [pltpu.async_copy(
                      x_sh.at[src_v.at[base + k, pl.ds(q * sr, sr)]],
                      rows_v.at[pl.ds((k * SUB + q) * sr, sr)], gsem)
                  for k in range(K) for q in range(SUB)]
            for g in gs:
                g.wait()
            ws = [pltpu.async_copy(
                      rows_v.at[pl.ds(k * CHUNK, CHUNK)],
                      out_hbm.at[pl.ds(ebase + (base + k) * CHUNK, CHUNK)],
                      wsem)
                  for k in range(K)]
            for w in ws:
                w.wait()
            return carry

        for s in range(NSTAGE):
            pltpu.sync_copy(srcw_hbm.at[wid, pl.ds(s * spc, spc)], src_v)
            lax.fori_loop(0, spc // K, body, 0)

    return sc_gather


def _make_sc_scatter(d, cpt, zr):
    aggr = NS * zr
    spc = cpt // NSTAGE
    ept = cpt * CHUNK
    mesh = plsc.VectorSubcoreMesh(core_axis_name="c", subcore_axis_name="s")

    @functools.partial(
        pl.kernel,
        out_type=jax.ShapeDtypeStruct((NC * aggr, d), jnp.float32),
        mesh=mesh,
        scratch_types=[
            pltpu.VMEM((spc, CHUNK), jnp.int32),
            pltpu.VMEM((K * CHUNK, d), jnp.float32),
            pltpu.VMEM_SHARED((aggr, d), jnp.float32),
            pltpu.SemaphoreType.DMA,
            pltpu.SemaphoreType.DMA,
        ],
    )
    def sc_scatter(rows_hbm, dstw_hbm, zero_hbm, out_hbm,
                   dst_v, rows_v, agg_sh, gsem, ssem):
        cid = lax.axis_index("c")
        sid = lax.axis_index("s")
        wid = cid * NS + sid

        pltpu.sync_copy(zero_hbm, rows_v.at[pl.ds(0, ZB)])
        zbase = sid * zr
        for off, ln in _row_chunks(zr):
            pltpu.sync_copy(rows_v.at[pl.ds(0, ln)],
                            agg_sh.at[pl.ds(zbase + off, ln)])
        plsc.subcore_barrier()

        ebase = wid * ept

        def body(p, carry):
            base = p * K
            gs = [pltpu.async_copy(
                      rows_hbm.at[pl.ds(ebase + (base + k) * CHUNK, CHUNK)],
                      rows_v.at[pl.ds(k * CHUNK, CHUNK)], gsem)
                  for k in range(K)]
            for g in gs:
                g.wait()
            ss = [pltpu.async_copy(rows_v.at[pl.ds(k * CHUNK, CHUNK)],
                                   agg_sh.at[dst_v.at[base + k]], ssem,
                                   add=True)
                  for k in range(K)]
            for s_ in ss:
                s_.wait()
            return carry

        for s in range(NSTAGE):
            pltpu.sync_copy(dstw_hbm.at[wid, pl.ds(s * spc, spc)], dst_v)
            lax.fori_loop(0, spc // K, body, 0)
        plsc.subcore_barrier()

        obase = cid * aggr + zbase
        for off, ln in _row_chunks(zr):
            pltpu.sync_copy(agg_sh.at[pl.ds(zbase + off, ln)],
                            rows_v.at[pl.ds(0, ln)])
            pltpu.sync_copy(rows_v.at[pl.ds(0, ln)],
                            out_hbm.at[pl.ds(obase + off, ln)])

    return sc_scatter


def _mlp_body(n, x_ref, p_ref, w1_ref, w1t_ref, b1_ref, gamma_ref, beta_ref,
              w2_ref, w2t_ref, b2_ref, u1_ref, u2_ref, out_ref):
    f32 = jnp.float32
    hi = lax.Precision.HIGHEST

    h = x_ref[...] + p_ref[0, :n, :] + p_ref[1, :n, :]

    u1 = u1_ref[...]
    v1 = jnp.dot(u1, w1_ref[...], precision=hi, preferred_element_type=f32)
    v1 = v1 / (jnp.sqrt(jnp.sum(v1 * v1)) + 1e-12)
    wv1 = jnp.dot(v1, w1t_ref[...], precision=hi, preferred_element_type=f32)
    sigma1 = jnp.sum(wv1 * wv1) / (jnp.sqrt(jnp.sum(wv1 * wv1)) + 1e-12)

    h1 = jnp.dot(h, w1t_ref[...], precision=hi, preferred_element_type=f32)
    h1 = h1 / sigma1 + b1_ref[...]
    h1 = jnp.maximum(h1, 0.0)

    mean = jnp.mean(h1, axis=0, keepdims=True)
    var = jnp.mean((h1 - mean) * (h1 - mean), axis=0, keepdims=True)
    hn = (h1 - mean) / jnp.sqrt(var + 1e-5) * gamma_ref[...] + beta_ref[...]

    u2 = u2_ref[...]
    v2 = jnp.dot(u2, w2_ref[...], precision=hi, preferred_element_type=f32)
    v2 = v2 / (jnp.sqrt(jnp.sum(v2 * v2)) + 1e-12)
    wv2 = jnp.dot(v2, w2t_ref[...], precision=hi, preferred_element_type=f32)
    sigma2 = jnp.sum(wv2 * wv2) / (jnp.sqrt(jnp.sum(wv2 * wv2)) + 1e-12)

    o = jnp.dot(hn, w2t_ref[...], precision=hi, preferred_element_type=f32)
    out_ref[...] = o / sigma2 + b2_ref[...]


def kernel(x, edge_index, W1, b1, u1, gamma, beta, W2, b2, u2):
    n, d = x.shape
    e = edge_index.shape[1]
    nhid = W1.shape[0]

    cpt = -(-e // (NW * CHUNK))
    cpt = -(-cpt // (NSTAGE * K)) * (NSTAGE * K)
    e_pad = NW * cpt * CHUNK
    zr = -(-(n + 1) // NS)
    zr = -(-zr // 8) * 8

    src = edge_index[0].astype(jnp.int32)
    dst = edge_index[1].astype(jnp.int32)
    pad = e_pad - e
    srcw = jnp.concatenate([src, jnp.zeros((pad,), jnp.int32)]).reshape(
        NW, cpt, CHUNK)
    dst_pad = n + jnp.arange(pad, dtype=jnp.int32) % jnp.int32(NS * zr - n)
    dstw = jnp.concatenate([dst, dst_pad]).reshape(NW, cpt, CHUNK)
    zero = jnp.zeros((ZB, d), jnp.float32)
    xp = jnp.concatenate([x, jnp.zeros((NS * zr - n, d), jnp.float32)])

    sc_gather = _make_sc_gather(d, cpt, zr)
    rows = sc_gather(xp, srcw)
    sc_scatter = _make_sc_scatter(d, cpt, zr)
    partials = sc_scatter(rows, dstw, zero)
    aggr = NS * zr
    p = partials.reshape(NC, aggr, d)

    vspec = pl.BlockSpec(memory_space=pltpu.VMEM)
    out = pl.pallas_call(
        functools.partial(_mlp_body, n),
        out_shape=jax.ShapeDtypeStruct((n, nhid), jnp.float32),
        in_specs=[vspec] * 12,
        out_specs=vspec,
    )(x, p, W1, W1.T, b1.reshape(1, nhid), gamma.reshape(1, nhid),
      beta.reshape(1, nhid), W2, W2.T, b2.reshape(1, nhid),
      u1.reshape(1, nhid), u2.reshape(1, nhid))
    return out
